# Initial kernel scaffold; baseline (speedup 1.0000x reference)
import jax
import jax.numpy as jnp
from jax.experimental import pallas as pl


def kernel(x, table):
    raise NotImplementedError("write your pallas kernel here")


# speedup vs baseline: 4.8088x; 4.8088x over previous
import functools

import jax
import jax.numpy as jnp
from jax import lax
from jax.experimental import pallas as pl
from jax.experimental.pallas import tpu as pltpu
from jax.experimental.pallas import tpu_sc as plsc

H_DIM = 32
ROW = 128


@functools.lru_cache(maxsize=None)
def _make_gather(n_rows, vocab, d, k):
    info = plsc.get_sparse_core_info()
    nc, ns = info.num_cores, info.num_subcores
    nw = nc * ns
    rows_per_w = n_rows // nw
    n_bursts = rows_per_w // k
    assert rows_per_w % k == 0 and n_rows % nw == 0

    mesh = plsc.VectorSubcoreMesh(core_axis_name="c", subcore_axis_name="s")

    @functools.partial(
        pl.kernel,
        mesh=mesh,
        compiler_params=pltpu.CompilerParams(use_tc_tiling_on_sc=False),
        out_type=jax.ShapeDtypeStruct((n_rows * ROW, d), jnp.float32),
        scratch_types=[
            pltpu.VMEM((k, ROW), jnp.int32),
            pltpu.VMEM((k * ROW, d), jnp.float32),
            pltpu.SemaphoreType.DMA,
        ],
    )
    def kern(idx_hbm, table_hbm, out_hbm, idx_v, rows_v, gsem):
        wid = lax.axis_index("s") * nc + lax.axis_index("c")
        row_base = wid * rows_per_w

        def burst(i, carry):
            r0 = row_base + i * k
            pltpu.sync_copy(idx_hbm.at[pl.ds(r0, k)], idx_v)
            copies = [
                pltpu.async_copy(
                    table_hbm.at[idx_v.at[j]],
                    rows_v.at[pl.ds(j * ROW, ROW)],
                    gsem,
                )
                for j in range(k)
            ]
            for c in copies:
                c.wait()
            pltpu.sync_copy(rows_v, out_hbm.at[pl.ds(r0 * ROW, k * ROW)])
            return carry

        lax.fori_loop(0, n_bursts, burst, 0)

    return kern


def kernel(x, table):
    b = x.size
    idx = x.reshape(b // ROW, ROW).astype(jnp.int32)
    out = _make_gather(b // ROW, table.shape[0], table.shape[1], 8)(idx, table)
    return out.reshape(x.shape + (table.shape[1],))


# speedup vs baseline: 5.0339x; 1.0468x over previous
import functools

import jax
import jax.numpy as jnp
from jax import lax
from jax.experimental import pallas as pl
from jax.experimental.pallas import tpu as pltpu
from jax.experimental.pallas import tpu_sc as plsc

H_DIM = 32
ROW = 128


@functools.lru_cache(maxsize=None)
def _make_gather(n_rows, vocab, d, k, nbuf):
    info = plsc.get_sparse_core_info()
    nc, ns = info.num_cores, info.num_subcores
    nw = nc * ns
    rows_per_w = n_rows // nw
    n_bursts = rows_per_w // k
    assert rows_per_w % k == 0 and n_rows % nw == 0 and n_bursts % nbuf == 0

    mesh = plsc.VectorSubcoreMesh(core_axis_name="c", subcore_axis_name="s")

    @functools.partial(
        pl.kernel,
        mesh=mesh,
        compiler_params=pltpu.CompilerParams(use_tc_tiling_on_sc=False),
        out_type=jax.ShapeDtypeStruct((n_rows * ROW, d), jnp.float32),
        scratch_types=[
            pltpu.VMEM((nbuf, k, ROW), jnp.int32),
            pltpu.VMEM((nbuf, k * ROW, d), jnp.float32),
            pltpu.SemaphoreType.DMA((nbuf,)),
            pltpu.SemaphoreType.DMA((nbuf,)),
            pltpu.SemaphoreType.DMA((nbuf,)),
        ],
    )
    def kern(idx_hbm, table_hbm, out_hbm, idx_v, rows_v, isem, gsem, osem):
        wid = lax.axis_index("s") * nc + lax.axis_index("c")
        row_base = wid * rows_per_w

        def idx_src(g):
            return idx_hbm.at[pl.ds(row_base + g * k, k)]

        def out_dst(g):
            return out_hbm.at[pl.ds((row_base + g * k) * ROW, k * ROW)]

        pltpu.async_copy(idx_src(0), idx_v.at[0], isem.at[0])

        def outer(i, carry):
            for b in range(nbuf):
                g = i * nbuf + b
                nb = (b + 1) % nbuf

                @pl.when(g + 1 < n_bursts)
                def _():
                    pltpu.async_copy(idx_src(g + 1), idx_v.at[nb], isem.at[nb])

                pltpu.make_async_copy(idx_src(g), idx_v.at[b], isem.at[b]).wait()

                @pl.when(i >= 1)
                def _():
                    pltpu.make_async_copy(
                        rows_v.at[b], out_dst(g - nbuf), osem.at[b]
                    ).wait()

                copies = [
                    pltpu.async_copy(
                        table_hbm.at[idx_v.at[b].at[j]],
                        rows_v.at[b].at[pl.ds(j * ROW, ROW)],
                        gsem.at[b],
                    )
                    for j in range(k)
                ]
                for c in copies:
                    c.wait()

                pltpu.async_copy(rows_v.at[b], out_dst(g), osem.at[b])
            return carry

        lax.fori_loop(0, n_bursts // nbuf, outer, 0)

        for b in range(nbuf):
            g = n_bursts - nbuf + b
            pltpu.make_async_copy(rows_v.at[b], out_dst(g), osem.at[b]).wait()

    return kern


def kernel(x, table):
    b = x.size
    idx = x.reshape(b // ROW, ROW).astype(jnp.int32)
    out = _make_gather(b // ROW, table.shape[0], table.shape[1], 8, 2)(idx, table)
    return out.reshape(x.shape + (table.shape[1],))
